# baseline (speedup 1.0000x reference)
import functools

import jax
import jax.numpy as jnp
from jax import lax
from jax.experimental import pallas as pl
from jax.experimental.pallas import tpu as pltpu
from jax.experimental.pallas import tpu_sc as plsc

NC = 2
NS = 16
L = 16
NW = NC * NS

B = 16384
D = 32
BPW = B // NW
CH = 128
NCH = BPW // CH
GROUPS = BPW // L


def _mf_body(uidx_hbm, iidx_hbm, ut_hbm, it_hbm, ubt_hbm, ibt_hbm, gb_hbm,
             out_hbm,
             uidx_v, iidx_v, urows_v, irows_v, ub_v, ib_v, gb_v, out_v, sem):
    wid = lax.axis_index("s") * NC + lax.axis_index("c")

    pltpu.sync_copy(uidx_hbm.at[wid], uidx_v)
    pltpu.sync_copy(iidx_hbm.at[wid], iidx_v)
    pltpu.sync_copy(gb_hbm, gb_v)

    copies = []
    for c in range(NCH):
        dst = pl.ds(c * CH, CH)
        copies.append(pltpu.async_copy(ut_hbm.at[uidx_v.at[c]],
                                       urows_v.at[dst], sem))
        copies.append(pltpu.async_copy(it_hbm.at[iidx_v.at[c]],
                                       irows_v.at[dst], sem))
        copies.append(pltpu.async_copy(ubt_hbm.at[uidx_v.at[c]],
                                       ub_v.at[dst], sem))
        copies.append(pltpu.async_copy(ibt_hbm.at[iidx_v.at[c]],
                                       ib_v.at[dst], sem))
    for cp in copies:
        cp.wait()

    gb = gb_v[...]
    lane = lax.iota(jnp.int32, L)
    zeros = jnp.zeros((L,), jnp.int32)

    def group(g, _):
        rows = g * L + lane
        acc = ub_v[pl.ds(g * L, L)] + ib_v[pl.ds(g * L, L)]
        cols = zeros
        ones = jnp.ones((L,), jnp.int32)
        for _d in range(D):
            u = plsc.load_gather(urows_v, [rows, cols])
            i = plsc.load_gather(irows_v, [rows, cols])
            acc = acc + u * i
            cols = cols + ones
        pred = acc + gb
        sig = 1.0 / (1.0 + jnp.exp(-pred))
        out_v[pl.ds(g * L, L)] = sig
        return 0

    lax.fori_loop(0, GROUPS, group, 0)

    pltpu.sync_copy(out_v, out_hbm.at[wid])


@jax.jit
def _mf_call(user_idx, item_idx, user_table, item_table,
             user_bias_table, item_bias_table, gb16):
    mesh = plsc.VectorSubcoreMesh(core_axis_name="c", subcore_axis_name="s",
                                  num_cores=NC, num_subcores=NS)
    fn = pl.kernel(
        _mf_body,
        out_type=jax.ShapeDtypeStruct((NW, BPW), jnp.float32),
        mesh=mesh,
        compiler_params=pltpu.CompilerParams(needs_layout_passes=False,
                                             use_tc_tiling_on_sc=False),
        scratch_types=[
            pltpu.VMEM((NCH, CH), jnp.int32),
            pltpu.VMEM((NCH, CH), jnp.int32),
            pltpu.VMEM((BPW, D), jnp.float32),
            pltpu.VMEM((BPW, D), jnp.float32),
            pltpu.VMEM((BPW,), jnp.float32),
            pltpu.VMEM((BPW,), jnp.float32),
            pltpu.VMEM((L,), jnp.float32),
            pltpu.VMEM((BPW,), jnp.float32),
            pltpu.SemaphoreType.DMA,
        ],
    )
    out = fn(user_idx.reshape(NW, NCH, CH), item_idx.reshape(NW, NCH, CH),
             user_table, item_table,
             user_bias_table.reshape(-1), item_bias_table.reshape(-1), gb16)
    return out.reshape(B)


def kernel(user_idx, item_idx, user_table, item_table,
           user_bias_table, item_bias_table, global_bias):
    gb16 = jnp.broadcast_to(global_bias.astype(jnp.float32), (L,))
    return _mf_call(user_idx.astype(jnp.int32), item_idx.astype(jnp.int32),
                    user_table, item_table,
                    user_bias_table, item_bias_table, gb16)
